# baseline (speedup 1.0000x reference)
import jax
import jax.numpy as jnp
from jax.experimental import pallas as pl
from jax.experimental.pallas import tpu as pltpu

G = 5
E = 5
J = 5
T = 9
D = 12
JT = G * J
IN = T * J * D
H1 = 512
H2 = 256
ZD = 64
B = 1024
BT = 512
NB = B // BT


def _moe_kernel(x_ref, W1_ref, b1_ref, W2_ref, b2_ref, Wmu_ref, bmu_ref,
                Wlv_ref, blv_ref, Wd1_ref, bd1_ref, Wd2_ref, bd2_ref,
                Wd3_ref, bd3_ref, mu_ref, lv_ref, xh_ref, idx_ref,
                slab_s, xf_s, sem_in, sem_out):
    bi = pl.program_id(0)
    g = pl.program_id(1)

    def in_cp(bb):
        return pltpu.make_async_copy(
            x_ref.at[:, :, :, pl.ds(bb * BT, BT)], slab_s.at[bb],
            sem_in.at[bb])

    def out_cp(bb):
        return pltpu.make_async_copy(
            slab_s.at[bb], xh_ref.at[:, :, :, pl.ds(bb * BT, BT)],
            sem_out.at[bb])

    @pl.when((bi == 0) & (g == 0))
    def _prologue():
        for bb in range(NB):
            in_cp(bb).start()
        in_cp(0).wait()
    for bb in range(1, NB):
        @pl.when((bi == bb) & (g == 0))
        def _():
            in_cp(bb).wait()

    for bb in range(NB):
        for gg in range(G):
            @pl.when((bi == bb) & (g == gg))
            def _():
                for t in range(T):
                    for j in range(J):
                        r = D * (J * t + j)
                        xf_s[r:r + D, :] = slab_s[bb, t, :, gg * J + j, :]

    xfb = xf_s[...].T

    def expert(e):
        h1 = jax.nn.relu(jnp.dot(xfb, W1_ref[0, e]) + b1_ref[0, e, 0])
        h2 = jax.nn.relu(jnp.dot(h1, W2_ref[0, e]) + b2_ref[0, e, 0])
        mu = jnp.dot(h2, Wmu_ref[0, e]) + bmu_ref[0, e, 0]
        lv = jnp.dot(h2, Wlv_ref[0, e]) + blv_ref[0, e, 0]
        d1 = jax.nn.relu(jnp.dot(mu, Wd1_ref[0, e]) + bd1_ref[0, e, 0])
        d2 = jax.nn.relu(jnp.dot(d1, Wd2_ref[0, e]) + bd2_ref[0, e, 0])
        xh = jnp.dot(d2, Wd3_ref[0, e]) + bd3_ref[0, e, 0]
        diff = xh - xfb
        err = jnp.mean(diff * diff, axis=-1, keepdims=True)
        return mu, lv, xh, err

    mu_b, lv_b, xh_b, err_b = expert(0)
    idx_b = jnp.zeros((BT, 1), dtype=jnp.int32)
    for e in range(1, E):
        mu_e, lv_e, xh_e, err_e = expert(e)
        better = err_e < err_b
        mu_b = jnp.where(better, mu_e, mu_b)
        lv_b = jnp.where(better, lv_e, lv_b)
        xh_b = jnp.where(better, xh_e, xh_b)
        idx_b = jnp.where(better, jnp.int32(e), idx_b)
        err_b = jnp.where(better, err_e, err_b)

    mu_ref[0] = mu_b.T
    lv_ref[0] = lv_b.T
    idx_ref[0] = jnp.broadcast_to(idx_b.T, (8, BT))

    xhT = xh_b.T
    for bb in range(NB):
        for gg in range(G):
            @pl.when((bi == bb) & (g == gg))
            def _():
                for t in range(T):
                    for j in range(J):
                        r = D * (J * t + j)
                        slab_s[bb, t, :, gg * J + j, :] = xhT[r:r + D, :]

    for bb in range(NB):
        @pl.when((bi == bb) & (g == G - 1))
        def _():
            out_cp(bb).start()
    @pl.when((bi == NB - 1) & (g == G - 1))
    def _drain():
        for bb in range(NB):
            out_cp(bb).wait()


def kernel(x, W1, b1, W2, b2, Wmu, bmu, Wlv, blv, Wd1, bd1, Wd2, bd2, Wd3, bd3):
    Bb = x.shape[0]
    nb = Bb // BT
    xt = x.transpose(1, 3, 2, 0)

    wspec = lambda *s: pl.BlockSpec((1, E) + s, lambda b, g: (g, 0) + (0,) * len(s))
    wspec1 = lambda *s: pl.BlockSpec((1, E) + s, lambda b, g: (g, 0) + (0,) * len(s),
                                     pipeline_mode=pl.Buffered(buffer_count=1))
    bias = lambda a: a.reshape(G, E, 1, a.shape[-1])
    bspec = lambda n: pl.BlockSpec((1, E, 1, n), lambda b, g: (g, 0, 0, 0))

    mu_t, lv_t, xh_t, idx = pl.pallas_call(
        _moe_kernel,
        grid=(nb, G),
        in_specs=[
            pl.BlockSpec(memory_space=pl.ANY),
            wspec(IN, H1), bspec(H1),
            wspec(H1, H2), bspec(H2),
            wspec1(H2, ZD), bspec(ZD),
            wspec1(H2, ZD), bspec(ZD),
            wspec1(ZD, H2), bspec(H2),
            wspec1(H2, H1), bspec(H1),
            wspec(H1, IN), bspec(IN),
        ],
        out_specs=[
            pl.BlockSpec((1, ZD, BT), lambda b, g: (g, 0, b)),
            pl.BlockSpec((1, ZD, BT), lambda b, g: (g, 0, b)),
            pl.BlockSpec(memory_space=pl.ANY),
            pl.BlockSpec((1, 8, BT), lambda b, g: (g, 0, b)),
        ],
        out_shape=[
            jax.ShapeDtypeStruct((G, ZD, Bb), jnp.float32),
            jax.ShapeDtypeStruct((G, ZD, Bb), jnp.float32),
            jax.ShapeDtypeStruct((T, D, JT, Bb), jnp.float32),
            jax.ShapeDtypeStruct((G, 8, Bb), jnp.int32),
        ],
        scratch_shapes=[
            pltpu.VMEM((NB, T, D, JT, BT), jnp.float32),
            pltpu.VMEM((IN, BT), jnp.float32),
            pltpu.SemaphoreType.DMA((NB,)),
            pltpu.SemaphoreType.DMA((NB,)),
        ],
    )(xt, W1, bias(b1), W2, bias(b2), Wmu, bias(bmu), Wlv, bias(blv),
      Wd1, bias(bd1), Wd2, bias(bd2), Wd3, bias(bd3))

    mu_sel = mu_t.transpose(0, 2, 1)
    lv_sel = lv_t.transpose(0, 2, 1)
    xhat = xh_t.transpose(3, 0, 2, 1)
    return mu_sel, lv_sel, xhat, idx[:, 0, :]


# speedup vs baseline: 1.0572x; 1.0572x over previous
import jax
import jax.numpy as jnp
from jax.experimental import pallas as pl
from jax.experimental.pallas import tpu as pltpu

G = 5
E = 5
J = 5
T = 9
D = 12
JT = G * J
IN = T * J * D
H1 = 512
H2 = 256
ZD = 64
B = 1024
BT = 512
NB = B // BT


def _moe_kernel(x_ref, W1_ref, b1_ref, W2_ref, b2_ref, Wmu_ref, bmu_ref,
                Wlv_ref, blv_ref, Wd1_ref, bd1_ref, Wd2_ref, bd2_ref,
                Wd3_ref, bd3_ref, mu_ref, lv_ref, xh_ref, idx_ref,
                slab_s, xf_s, sem_in, sem_out):
    bi = pl.program_id(0)
    g = pl.program_id(1)

    @pl.when(g == 0)
    def _load():
        cp = pltpu.make_async_copy(
            x_ref.at[:, :, :, pl.ds(bi * BT, BT)], slab_s, sem_in)
        cp.start()
        cp.wait()

    for gg in range(G):
        @pl.when(g == gg)
        def _():
            for t in range(T):
                for j in range(J):
                    r = D * (J * t + j)
                    xf_s[r:r + D, :] = slab_s[t, :, gg * J + j, :]

    xfb = xf_s[...].T

    def expert(e):
        h1 = jax.nn.relu(jnp.dot(xfb, W1_ref[0, e]) + b1_ref[0, e, 0])
        h2 = jax.nn.relu(jnp.dot(h1, W2_ref[0, e]) + b2_ref[0, e, 0])
        mu = jnp.dot(h2, Wmu_ref[0, e]) + bmu_ref[0, e, 0]
        lv = jnp.dot(h2, Wlv_ref[0, e]) + blv_ref[0, e, 0]
        d1 = jax.nn.relu(jnp.dot(mu, Wd1_ref[0, e]) + bd1_ref[0, e, 0])
        d2 = jax.nn.relu(jnp.dot(d1, Wd2_ref[0, e]) + bd2_ref[0, e, 0])
        xh = jnp.dot(d2, Wd3_ref[0, e]) + bd3_ref[0, e, 0]
        diff = xh - xfb
        err = jnp.mean(diff * diff, axis=-1, keepdims=True)
        return mu, lv, xh, err

    mu_b, lv_b, xh_b, err_b = expert(0)
    idx_b = jnp.zeros((BT, 1), dtype=jnp.int32)
    for e in range(1, E):
        mu_e, lv_e, xh_e, err_e = expert(e)
        better = err_e < err_b
        mu_b = jnp.where(better, mu_e, mu_b)
        lv_b = jnp.where(better, lv_e, lv_b)
        xh_b = jnp.where(better, xh_e, xh_b)
        idx_b = jnp.where(better, jnp.int32(e), idx_b)
        err_b = jnp.where(better, err_e, err_b)

    mu_ref[0] = mu_b.T
    lv_ref[0] = lv_b.T
    idx_ref[0] = jnp.broadcast_to(idx_b.T, (8, BT))

    xhT = xh_b.T
    for gg in range(G):
        @pl.when(g == gg)
        def _():
            for t in range(T):
                for j in range(J):
                    r = D * (J * t + j)
                    slab_s[t, :, gg * J + j, :] = xhT[r:r + D, :]

    @pl.when(g == G - 1)
    def _flush():
        cp = pltpu.make_async_copy(
            slab_s, xh_ref.at[:, :, :, pl.ds(bi * BT, BT)], sem_out)
        cp.start()
        cp.wait()


def kernel(x, W1, b1, W2, b2, Wmu, bmu, Wlv, blv, Wd1, bd1, Wd2, bd2, Wd3, bd3):
    Bb = x.shape[0]
    nb = Bb // BT
    xt = x.transpose(1, 3, 2, 0)

    wspec = lambda *s: pl.BlockSpec((1, E) + s, lambda b, g: (g, 0) + (0,) * len(s))
    bias = lambda a: a.reshape(G, E, 1, a.shape[-1])
    bspec = lambda n: pl.BlockSpec((1, E, 1, n), lambda b, g: (g, 0, 0, 0))

    mu_t, lv_t, xh_t, idx = pl.pallas_call(
        _moe_kernel,
        grid=(nb, G),
        in_specs=[
            pl.BlockSpec(memory_space=pl.ANY),
            wspec(IN, H1), bspec(H1),
            wspec(H1, H2), bspec(H2),
            wspec(H2, ZD), bspec(ZD),
            wspec(H2, ZD), bspec(ZD),
            wspec(ZD, H2), bspec(H2),
            wspec(H2, H1), bspec(H1),
            wspec(H1, IN), bspec(IN),
        ],
        out_specs=[
            pl.BlockSpec((1, ZD, BT), lambda b, g: (g, 0, b)),
            pl.BlockSpec((1, ZD, BT), lambda b, g: (g, 0, b)),
            pl.BlockSpec(memory_space=pl.ANY),
            pl.BlockSpec((1, 8, BT), lambda b, g: (g, 0, b)),
        ],
        out_shape=[
            jax.ShapeDtypeStruct((G, ZD, Bb), jnp.float32),
            jax.ShapeDtypeStruct((G, ZD, Bb), jnp.float32),
            jax.ShapeDtypeStruct((T, D, JT, Bb), jnp.float32),
            jax.ShapeDtypeStruct((G, 8, Bb), jnp.int32),
        ],
        scratch_shapes=[
            pltpu.VMEM((T, D, JT, BT), jnp.float32),
            pltpu.VMEM((IN, BT), jnp.float32),
            pltpu.SemaphoreType.DMA,
            pltpu.SemaphoreType.DMA,
        ],
    )(xt, W1, bias(b1), W2, bias(b2), Wmu, bias(bmu), Wlv, bias(blv),
      Wd1, bias(bd1), Wd2, bias(bd2), Wd3, bias(bd3))

    mu_sel = mu_t.transpose(0, 2, 1)
    lv_sel = lv_t.transpose(0, 2, 1)
    xhat = xh_t.transpose(3, 0, 2, 1)
    return mu_sel, lv_sel, xhat, idx[:, 0, :]
